# baseline (speedup 1.0000x reference)
import functools

import jax
import jax.numpy as jnp
from jax import lax
from jax.experimental import pallas as pl
from jax.experimental.pallas import tpu as pltpu
from jax.experimental.pallas import tpu_sc as plsc

INVALID_TOK = -1
NO_SPEC = -2
LANES = 16
NUM_WORKERS = 32


def _sc_body(vocab, rows_per_w, logits_hbm, spec_hbm, out_hbm,
             rowbuf, specbuf, outbuf, active_sm):
    wid = lax.axis_index("s") * 2 + lax.axis_index("c")
    base_row = wid * rows_per_w
    lane = lax.iota(jnp.int32, LANES)

    pltpu.sync_copy(spec_hbm.at[pl.ds(base_row, LANES)], specbuf)
    outbuf[...] = jnp.full((LANES,), INVALID_TOK, jnp.int32)

    def step(t, carry):
        @pl.when((t & 7) == 0)
        def _():
            active_sm[0] = jnp.int32(1)

        @pl.when(active_sm[0] == 1)
        def _():
            row = base_row + t
            pltpu.sync_copy(logits_hbm.at[pl.ds(row * vocab, vocab)], rowbuf)

            def inner(i, c):
                cm, ci, pos = c
                v = rowbuf[pl.ds(i * LANES, LANES)]
                gt = v > cm
                return (jnp.where(gt, v, cm), jnp.where(gt, pos, ci),
                        pos + LANES)

            cm0 = jnp.full((LANES,), -jnp.inf, jnp.float32)
            ci0 = jnp.zeros((LANES,), jnp.int32)
            cm, ci, _ = lax.fori_loop(0, vocab // LANES, inner,
                                      (cm0, ci0, lane))
            m = jnp.max(cm)
            cand = jnp.where(cm == m, ci, jnp.int32(2**31 - 1))
            tok = jnp.min(cand)

            sel = lane == t
            bct = jnp.full((LANES,), tok, jnp.int32)
            outbuf[...] = jnp.where(sel, bct, outbuf[...])
            ok = jnp.any(sel & (bct == specbuf[...]))
            active_sm[0] = jnp.where(ok, 1, 0).astype(jnp.int32)

        return carry

    lax.fori_loop(0, rows_per_w, step, jnp.int32(0))
    pltpu.sync_copy(outbuf, out_hbm.at[pl.ds(base_row, LANES)])


def kernel(logits, spec_token_ids):
    batch, spec_len = spec_token_ids.shape
    sample_len = spec_len + 1
    rows, vocab = logits.shape
    rows_per_w = rows // NUM_WORKERS

    spec_pad = jnp.concatenate(
        [spec_token_ids.astype(jnp.int32),
         jnp.full((batch, 1), NO_SPEC, jnp.int32)], axis=1).reshape(-1)

    mesh = plsc.VectorSubcoreMesh(core_axis_name="c", subcore_axis_name="s")
    run = pl.kernel(
        functools.partial(_sc_body, vocab, rows_per_w),
        mesh=mesh,
        compiler_params=pltpu.CompilerParams(needs_layout_passes=False),
        out_type=jax.ShapeDtypeStruct((rows,), jnp.int32),
        scratch_types=[
            pltpu.VMEM((vocab,), jnp.float32),
            pltpu.VMEM((LANES,), jnp.int32),
            pltpu.VMEM((LANES,), jnp.int32),
            pltpu.SMEM((1,), jnp.int32),
        ],
    )
    out = run(logits.reshape(-1), spec_pad)
    return out.reshape(batch, sample_len)


# speedup vs baseline: 1.0941x; 1.0941x over previous
import functools

import jax
import jax.numpy as jnp
from jax import lax
from jax.experimental import pallas as pl
from jax.experimental.pallas import tpu as pltpu
from jax.experimental.pallas import tpu_sc as plsc

INVALID_TOK = -1
NO_SPEC = -2
LANES = 16
NUM_WORKERS = 32
N_SLOT = 5
N_CHUNK = 5
STRIDE = N_SLOT * LANES


def _argmax_row(row, vocab, logits_hbm, bufs, sems, lane):
    chunk_w = vocab // N_CHUNK
    iters_per_chunk = chunk_w // STRIDE
    base = row * vocab
    neg = jnp.full((LANES,), -jnp.inf, jnp.float32)
    zero = jnp.zeros((LANES,), jnp.int32)
    carry = ((neg,) * N_SLOT, (zero,) * N_SLOT)

    cp = pltpu.async_copy(logits_hbm.at[pl.ds(base, chunk_w)], bufs[0],
                          sems[0])
    for c in range(N_CHUNK):
        nxt = None
        if c + 1 < N_CHUNK:
            nxt = pltpu.async_copy(
                logits_hbm.at[pl.ds(base + (c + 1) * chunk_w, chunk_w)],
                bufs[(c + 1) % 2], sems[(c + 1) % 2])
        cp.wait()
        buf = bufs[c % 2]
        lo = c * iters_per_chunk

        @plsc.parallel_loop(lo, lo + iters_per_chunk, unroll=2, carry=carry)
        def carry(i, cr, buf=buf, lo=lo):
            cms, cis = cr
            bi = jnp.full((LANES,), i, jnp.int32)
            off = i * STRIDE - lo * STRIDE
            ncm, nci = [], []
            for k in range(N_SLOT):
                v = buf[pl.ds(off + k * LANES, LANES)]
                gt = v > cms[k]
                ncm.append(jnp.where(gt, v, cms[k]))
                nci.append(jnp.where(gt, bi, cis[k]))
            return (tuple(ncm), tuple(nci))

        cp = nxt

    cms, cis = carry
    best_v, best_i = cms[0], cis[0] * STRIDE + lane
    for k in range(1, N_SLOT):
        idx = cis[k] * STRIDE + (k * LANES) + lane
        gt = (cms[k] > best_v) | ((cms[k] == best_v) & (idx < best_i))
        best_v = jnp.where(gt, cms[k], best_v)
        best_i = jnp.where(gt, idx, best_i)
    m = jnp.max(best_v)
    return jnp.min(jnp.where(best_v == m, best_i, jnp.int32(2**31 - 1)))


def _sc_body(vocab, rows_per_w, logits_hbm, spec_hbm, out_hbm,
             buf0, buf1, specbuf, outbuf, sem0, sem1, active_sm):
    wid = lax.axis_index("s") * 2 + lax.axis_index("c")
    base_row = wid * rows_per_w
    lane = lax.iota(jnp.int32, LANES)

    pltpu.sync_copy(spec_hbm.at[pl.ds(base_row, LANES)], specbuf)
    outbuf[...] = jnp.full((LANES,), INVALID_TOK, jnp.int32)

    def step(t, carry):
        @pl.when((t & 7) == 0)
        def _():
            active_sm[0] = jnp.int32(1)

        @pl.when(active_sm[0] == 1)
        def _():
            tok = _argmax_row(base_row + t, vocab, logits_hbm,
                              (buf0, buf1), (sem0, sem1), lane)
            sel = lane == t
            bct = jnp.full((LANES,), tok, jnp.int32)
            outbuf[...] = jnp.where(sel, bct, outbuf[...])
            ok = jnp.any(sel & (bct == specbuf[...]))
            active_sm[0] = jnp.where(ok, 1, 0).astype(jnp.int32)

        return carry

    lax.fori_loop(0, rows_per_w, step, jnp.int32(0))
    pltpu.sync_copy(outbuf, out_hbm.at[pl.ds(base_row, LANES)])


def kernel(logits, spec_token_ids):
    batch, spec_len = spec_token_ids.shape
    sample_len = spec_len + 1
    rows, vocab = logits.shape
    rows_per_w = rows // NUM_WORKERS
    chunk_w = vocab // N_CHUNK

    spec_pad = jnp.concatenate(
        [spec_token_ids.astype(jnp.int32),
         jnp.full((batch, 1), NO_SPEC, jnp.int32)], axis=1).reshape(-1)

    mesh = plsc.VectorSubcoreMesh(core_axis_name="c", subcore_axis_name="s")
    run = pl.kernel(
        functools.partial(_sc_body, vocab, rows_per_w),
        mesh=mesh,
        compiler_params=pltpu.CompilerParams(needs_layout_passes=False),
        out_type=jax.ShapeDtypeStruct((rows,), jnp.int32),
        scratch_types=[
            pltpu.VMEM((chunk_w,), jnp.float32),
            pltpu.VMEM((chunk_w,), jnp.float32),
            pltpu.VMEM((LANES,), jnp.int32),
            pltpu.VMEM((LANES,), jnp.int32),
            pltpu.SemaphoreType.DMA,
            pltpu.SemaphoreType.DMA,
            pltpu.SMEM((1,), jnp.int32),
        ],
    )
    out = run(logits.reshape(-1), spec_pad)
    return out.reshape(batch, sample_len)


# speedup vs baseline: 2.3795x; 2.1749x over previous
import functools

import jax
import jax.numpy as jnp
from jax import lax
from jax.experimental import pallas as pl
from jax.experimental.pallas import tpu as pltpu
from jax.experimental.pallas import tpu_sc as plsc

INVALID_TOK = -1
NO_SPEC = -2
LANES = 16
NUM_WORKERS = 32
N_SLOT = 4
STRIDE = N_SLOT * LANES
TILE_W = 128


def _lex_merge(best_v, best_i, v, idx):
    gt = (v > best_v) | ((v == best_v) & (idx < best_i))
    return jnp.where(gt, v, best_v), jnp.where(gt, idx, best_i)


def _argmax_row(rowbuf, tailbuf, rowmod, main_w, tail_w, lane):
    neg = jnp.full((LANES,), -jnp.inf, jnp.float32)
    zero = jnp.zeros((LANES,), jnp.int32)

    @plsc.parallel_loop(0, main_w // STRIDE, unroll=2,
                        carry=((neg,) * N_SLOT, (zero,) * N_SLOT))
    def carry(i, cr):
        cms, cis = cr
        bi = jnp.full((LANES,), i, jnp.int32)
        off = i * STRIDE
        ncm, nci = [], []
        for k in range(N_SLOT):
            v = rowbuf[0, pl.ds(off + k * LANES, LANES)]
            gt = v > cms[k]
            ncm.append(jnp.where(gt, v, cms[k]))
            nci.append(jnp.where(gt, bi, cis[k]))
        return (tuple(ncm), tuple(nci))

    cms, cis = carry
    best_v, best_i = cms[0], cis[0] * STRIDE + lane
    for k in range(1, N_SLOT):
        best_v, best_i = _lex_merge(best_v, best_i, cms[k],
                                    cis[k] * STRIDE + (k * LANES) + lane)
    for m in range(tail_w // LANES):
        best_v, best_i = _lex_merge(best_v, best_i,
                                    tailbuf[rowmod, pl.ds(m * LANES, LANES)],
                                    main_w + m * LANES + lane)
    gm = jnp.max(best_v)
    return jnp.min(jnp.where(best_v == gm, best_i, jnp.int32(2**31 - 1)))


def _sc_body(main_w, tail_w, rows_per_w, logits_hbm, spec_hbm, out_hbm,
             rowbuf, tailbuf, idxbuf, specbuf, outbuf, sem, active_sm):
    wid = lax.axis_index("s") * 2 + lax.axis_index("c")
    base_row = wid * rows_per_w
    lane = lax.iota(jnp.int32, LANES)

    pltpu.sync_copy(spec_hbm.at[pl.ds(base_row, LANES)], specbuf)
    outbuf[...] = jnp.full((LANES,), INVALID_TOK, jnp.int32)

    def step(t, carry):
        @pl.when((t & 7) == 0)
        def _():
            active_sm[0] = jnp.int32(1)
            grp = pl.multiple_of(base_row + t, 8)
            pltpu.sync_copy(
                logits_hbm.at[pl.ds(grp, 8), pl.ds(main_w, tail_w)],
                tailbuf)

        @pl.when(active_sm[0] == 1)
        def _():
            idxbuf[...] = jnp.full((LANES,), base_row + t, jnp.int32)
            pltpu.async_copy(
                logits_hbm.at[idxbuf.at[pl.ds(0, 1)], pl.ds(0, main_w)],
                rowbuf, sem).wait()
            tok = _argmax_row(rowbuf, tailbuf, t & 7, main_w, tail_w, lane)
            sel = lane == t
            bct = jnp.full((LANES,), tok, jnp.int32)
            outbuf[...] = jnp.where(sel, bct, outbuf[...])
            ok = jnp.any(sel & (bct == specbuf[...]))
            active_sm[0] = jnp.where(ok, 1, 0).astype(jnp.int32)

        return carry

    lax.fori_loop(0, rows_per_w, step, jnp.int32(0))
    pltpu.sync_copy(outbuf, out_hbm.at[pl.ds(base_row, LANES)])


def kernel(logits, spec_token_ids):
    batch, spec_len = spec_token_ids.shape
    sample_len = spec_len + 1
    rows, vocab = logits.shape
    rows_per_w = rows // NUM_WORKERS
    main_w = (vocab // TILE_W) * TILE_W
    tail_w = vocab - main_w

    spec_pad = jnp.concatenate(
        [spec_token_ids.astype(jnp.int32),
         jnp.full((batch, 1), NO_SPEC, jnp.int32)], axis=1).reshape(-1)

    mesh = plsc.VectorSubcoreMesh(core_axis_name="c", subcore_axis_name="s")
    run = pl.kernel(
        functools.partial(_sc_body, main_w, tail_w, rows_per_w),
        mesh=mesh,
        compiler_params=pltpu.CompilerParams(needs_layout_passes=False),
        out_type=jax.ShapeDtypeStruct((rows,), jnp.int32),
        scratch_types=[
            pltpu.VMEM((1, main_w), jnp.float32),
            pltpu.VMEM((8, tail_w), jnp.float32),
            pltpu.VMEM((LANES,), jnp.int32),
            pltpu.VMEM((LANES,), jnp.int32),
            pltpu.VMEM((LANES,), jnp.int32),
            pltpu.SemaphoreType.DMA,
            pltpu.SMEM((1,), jnp.int32),
        ],
    )
    out = run(logits, spec_pad)
    return out.reshape(batch, sample_len)


# speedup vs baseline: 5.5010x; 2.3119x over previous
import functools

import jax
import jax.numpy as jnp
from jax import lax
from jax.experimental import pallas as pl
from jax.experimental.pallas import tpu as pltpu
from jax.experimental.pallas import tpu_sc as plsc

INVALID_TOK = -1
LANES = 16
NUM_WORKERS = 32
BIG_I32 = 2**31 - 1

SC_V = 38400
SC_PER_W = SC_V // NUM_WORKERS
SC_CHUNK = 120
SC_N_CHUNK = SC_PER_W // SC_CHUNK
TC_BLOCK = 1400


def _sc_body(tc_v, n_rows, lt_hbm, out_val_hbm, out_idx_hbm,
             buf, acc_val, acc_idx, sem0, sem1):
    w = lax.axis_index("s") * 2 + lax.axis_index("c")
    vbase = tc_v + w * SC_PER_W
    lane = lax.iota(jnp.int32, LANES)
    del lane

    for j in range(n_rows // LANES):
        acc_val[pl.ds(j * LANES, LANES)] = jnp.full((LANES,), -jnp.inf,
                                                    jnp.float32)
        acc_idx[pl.ds(j * LANES, LANES)] = jnp.zeros((LANES,), jnp.int32)

    def start_copy(c, par):
        return pltpu.async_copy(
            lt_hbm.at[pl.ds(vbase + c * SC_CHUNK, SC_CHUNK), pl.ds(0, n_rows)],
            buf.at[par], sem0 if par == 0 else sem1)

    cp = start_copy(0, 0)
    for c in range(SC_N_CHUNK):
        nxt = None
        if c + 1 < SC_N_CHUNK:
            nxt = start_copy(c + 1, (c + 1) % 2)
        cp.wait()
        par = c % 2
        cbase = vbase + c * SC_CHUNK

        def jbody(j2, _, par=par, cbase=cbase):
            o = j2 * (2 * LANES)
            av0 = acc_val[pl.ds(o, LANES)]
            ai0 = acc_idx[pl.ds(o, LANES)]
            av1 = acc_val[pl.ds(o + LANES, LANES)]
            ai1 = acc_idx[pl.ds(o + LANES, LANES)]

            @plsc.parallel_loop(0, SC_CHUNK, unroll=2,
                                carry=(av0, ai0, av1, ai1))
            def carry(v, cr, par=par, o=o, cbase=cbase):
                av0, ai0, av1, ai1 = cr
                bi = jnp.full((LANES,), cbase + v, jnp.int32)
                x0 = buf[par, v, pl.ds(o, LANES)]
                x1 = buf[par, v, pl.ds(o + LANES, LANES)]
                g0 = x0 > av0
                g1 = x1 > av1
                return (jnp.where(g0, x0, av0), jnp.where(g0, bi, ai0),
                        jnp.where(g1, x1, av1), jnp.where(g1, bi, ai1))

            av0, ai0, av1, ai1 = carry
            acc_val[pl.ds(o, LANES)] = av0
            acc_idx[pl.ds(o, LANES)] = ai0
            acc_val[pl.ds(o + LANES, LANES)] = av1
            acc_idx[pl.ds(o + LANES, LANES)] = ai1
            return _

        lax.fori_loop(0, n_rows // (2 * LANES), jbody, jnp.int32(0))
        cp = nxt

    pltpu.sync_copy(acc_val, out_val_hbm.at[pl.ds(w * n_rows, n_rows)])
    pltpu.sync_copy(acc_idx, out_idx_hbm.at[pl.ds(w * n_rows, n_rows)])


def _tc_partial_body(n_blocks, x_ref, val_ref, idx_ref, acc_val, acc_idx):
    v = pl.program_id(0)
    x = x_ref[...]
    iota0 = lax.broadcasted_iota(jnp.int32, x.shape, 0)
    mx = jnp.max(x, axis=0)
    eq = x == mx[None, :]
    li = jnp.min(jnp.where(eq, iota0, BIG_I32), axis=0) + v * TC_BLOCK

    @pl.when(v == 0)
    def _():
        acc_val[0, :] = mx
        acc_idx[0, :] = li

    @pl.when(v > 0)
    def _():
        av = acc_val[0, :]
        gt = mx > av
        acc_val[0, :] = jnp.where(gt, mx, av)
        acc_idx[0, :] = jnp.where(gt, li, acc_idx[0, :])

    @pl.when(v == n_blocks - 1)
    def _():
        val_ref[...] = acc_val[0, :]
        idx_ref[...] = acc_idx[0, :]


def _combine_body(n_rows, spec_len, tcv_ref, tci_ref, scv_ref, sci_ref,
                  spec_ref, out_ref):
    bv = tcv_ref[...]
    bi = tci_ref[...]
    for s in range(NUM_WORKERS):
        v = scv_ref[pl.ds(s * n_rows, n_rows)]
        i = sci_ref[pl.ds(s * n_rows, n_rows)]
        gt = v > bv
        bv = jnp.where(gt, v, bv)
        bi = jnp.where(gt, i, bi)

    pos = lax.broadcasted_iota(jnp.int32, (n_rows,), 0)
    j = pos & (spec_len)
    mm = ((bi != spec_ref[...]) & (j != spec_len)).astype(jnp.int32)
    e = jnp.roll(mm, 1) * (j >= 1).astype(jnp.int32)
    e = e | (jnp.roll(e, 2) * (j >= 2).astype(jnp.int32))
    e = e | (jnp.roll(e, 4) * (j >= 4).astype(jnp.int32))
    out_ref[...] = jnp.where(e > 0, INVALID_TOK, bi)


def kernel(logits, spec_token_ids):
    batch, spec_len = spec_token_ids.shape
    sample_len = spec_len + 1
    n_rows, vocab = logits.shape
    tc_v = vocab - SC_V
    n_blocks = tc_v // TC_BLOCK
    assert n_blocks * TC_BLOCK == tc_v

    lt = logits.T

    spec_pad = jnp.concatenate(
        [spec_token_ids.astype(jnp.int32),
         jnp.full((batch, 1), -2, jnp.int32)], axis=1).reshape(-1)

    mesh = plsc.VectorSubcoreMesh(core_axis_name="c", subcore_axis_name="s")
    sc_run = pl.kernel(
        functools.partial(_sc_body, tc_v, n_rows),
        mesh=mesh,
        compiler_params=pltpu.CompilerParams(needs_layout_passes=False),
        out_type=(jax.ShapeDtypeStruct((NUM_WORKERS * n_rows,), jnp.float32),
                  jax.ShapeDtypeStruct((NUM_WORKERS * n_rows,), jnp.int32)),
        scratch_types=[
            pltpu.VMEM((2, SC_CHUNK, n_rows), jnp.float32),
            pltpu.VMEM((n_rows,), jnp.float32),
            pltpu.VMEM((n_rows,), jnp.int32),
            pltpu.SemaphoreType.DMA,
            pltpu.SemaphoreType.DMA,
        ],
    )
    sc_val, sc_idx = sc_run(lt)

    tc_val, tc_idx = pl.pallas_call(
        functools.partial(_tc_partial_body, n_blocks),
        grid=(n_blocks,),
        in_specs=[pl.BlockSpec((TC_BLOCK, n_rows), lambda v: (v, 0))],
        out_specs=[pl.BlockSpec((n_rows,), lambda v: (0,)),
                   pl.BlockSpec((n_rows,), lambda v: (0,))],
        out_shape=[jax.ShapeDtypeStruct((n_rows,), jnp.float32),
                   jax.ShapeDtypeStruct((n_rows,), jnp.int32)],
        scratch_shapes=[pltpu.VMEM((1, n_rows), jnp.float32),
                        pltpu.VMEM((1, n_rows), jnp.int32)],
        compiler_params=pltpu.CompilerParams(
            dimension_semantics=("arbitrary",)),
    )(lt)

    out = pl.pallas_call(
        functools.partial(_combine_body, n_rows, spec_len),
        out_shape=jax.ShapeDtypeStruct((n_rows,), jnp.int32),
    )(tc_val, tc_idx, sc_val, sc_idx, spec_pad)
    return out.reshape(batch, sample_len)


# speedup vs baseline: 5.6314x; 1.0237x over previous
import functools

import jax
import jax.numpy as jnp
from jax import lax
from jax.experimental import pallas as pl
from jax.experimental.pallas import tpu as pltpu
from jax.experimental.pallas import tpu_sc as plsc

INVALID_TOK = -1
LANES = 16
NUM_WORKERS = 32
BIG_I32 = 2**31 - 1

SC_V = 46080
SC_PER_W = SC_V // NUM_WORKERS
SC_CHUNK = 96
SC_N_CHUNK = SC_PER_W // SC_CHUNK
TC_BLOCK = 2696


def _sc_body(tc_v, n_rows, lt_hbm, out_val_hbm, out_idx_hbm,
             buf, acc_val, acc_idx, sem0, sem1):
    w = lax.axis_index("s") * 2 + lax.axis_index("c")
    vbase = tc_v + w * SC_PER_W
    lane = lax.iota(jnp.int32, LANES)
    del lane

    for j in range(n_rows // LANES):
        acc_val[pl.ds(j * LANES, LANES)] = jnp.full((LANES,), -jnp.inf,
                                                    jnp.float32)
        acc_idx[pl.ds(j * LANES, LANES)] = jnp.zeros((LANES,), jnp.int32)

    def start_copy(c, par):
        return pltpu.async_copy(
            lt_hbm.at[pl.ds(vbase + c * SC_CHUNK, SC_CHUNK), pl.ds(0, n_rows)],
            buf.at[par], sem0 if par == 0 else sem1)

    cp = start_copy(0, 0)
    for c in range(SC_N_CHUNK):
        nxt = None
        if c + 1 < SC_N_CHUNK:
            nxt = start_copy(c + 1, (c + 1) % 2)
        cp.wait()
        par = c % 2
        cbase = vbase + c * SC_CHUNK

        def jbody(j2, _, par=par, cbase=cbase):
            o = j2 * (2 * LANES)
            av0 = acc_val[pl.ds(o, LANES)]
            ai0 = acc_idx[pl.ds(o, LANES)]
            av1 = acc_val[pl.ds(o + LANES, LANES)]
            ai1 = acc_idx[pl.ds(o + LANES, LANES)]

            @plsc.parallel_loop(0, SC_CHUNK, unroll=2,
                                carry=(av0, ai0, av1, ai1))
            def carry(v, cr, par=par, o=o, cbase=cbase):
                av0, ai0, av1, ai1 = cr
                bi = jnp.full((LANES,), cbase + v, jnp.int32)
                x0 = buf[par, v, pl.ds(o, LANES)]
                x1 = buf[par, v, pl.ds(o + LANES, LANES)]
                g0 = x0 > av0
                g1 = x1 > av1
                return (jnp.where(g0, x0, av0), jnp.where(g0, bi, ai0),
                        jnp.where(g1, x1, av1), jnp.where(g1, bi, ai1))

            av0, ai0, av1, ai1 = carry
            acc_val[pl.ds(o, LANES)] = av0
            acc_idx[pl.ds(o, LANES)] = ai0
            acc_val[pl.ds(o + LANES, LANES)] = av1
            acc_idx[pl.ds(o + LANES, LANES)] = ai1
            return _

        lax.fori_loop(0, n_rows // (2 * LANES), jbody, jnp.int32(0))
        cp = nxt

    pltpu.sync_copy(acc_val, out_val_hbm.at[pl.ds(w * n_rows, n_rows)])
    pltpu.sync_copy(acc_idx, out_idx_hbm.at[pl.ds(w * n_rows, n_rows)])


def _tc_partial_body(n_blocks, x_ref, val_ref, idx_ref, acc_val, acc_idx):
    v = pl.program_id(0)
    x = x_ref[...]
    iota0 = lax.broadcasted_iota(jnp.int32, x.shape, 0)
    mx = jnp.max(x, axis=0)
    eq = x == mx[None, :]
    li = jnp.min(jnp.where(eq, iota0, BIG_I32), axis=0) + v * TC_BLOCK

    @pl.when(v == 0)
    def _():
        acc_val[0, :] = mx
        acc_idx[0, :] = li

    @pl.when(v > 0)
    def _():
        av = acc_val[0, :]
        gt = mx > av
        acc_val[0, :] = jnp.where(gt, mx, av)
        acc_idx[0, :] = jnp.where(gt, li, acc_idx[0, :])

    @pl.when(v == n_blocks - 1)
    def _():
        val_ref[...] = acc_val[0, :]
        idx_ref[...] = acc_idx[0, :]


def _combine_body(n_rows, spec_len, tcv_ref, tci_ref, scv_ref, sci_ref,
                  spec_ref, out_ref):
    bv = tcv_ref[...]
    bi = tci_ref[...]
    for s in range(NUM_WORKERS):
        v = scv_ref[pl.ds(s * n_rows, n_rows)]
        i = sci_ref[pl.ds(s * n_rows, n_rows)]
        gt = v > bv
        bv = jnp.where(gt, v, bv)
        bi = jnp.where(gt, i, bi)

    pos = lax.broadcasted_iota(jnp.int32, (n_rows,), 0)
    j = pos & (spec_len)
    mm = ((bi != spec_ref[...]) & (j != spec_len)).astype(jnp.int32)
    e = jnp.roll(mm, 1) * (j >= 1).astype(jnp.int32)
    e = e | (jnp.roll(e, 2) * (j >= 2).astype(jnp.int32))
    e = e | (jnp.roll(e, 4) * (j >= 4).astype(jnp.int32))
    out_ref[...] = jnp.where(e > 0, INVALID_TOK, bi)


def kernel(logits, spec_token_ids):
    batch, spec_len = spec_token_ids.shape
    sample_len = spec_len + 1
    n_rows, vocab = logits.shape
    tc_v = vocab - SC_V
    n_blocks = tc_v // TC_BLOCK
    assert n_blocks * TC_BLOCK == tc_v

    lt = logits.T

    spec_pad = jnp.concatenate(
        [spec_token_ids.astype(jnp.int32),
         jnp.full((batch, 1), -2, jnp.int32)], axis=1).reshape(-1)

    mesh = plsc.VectorSubcoreMesh(core_axis_name="c", subcore_axis_name="s")
    sc_run = pl.kernel(
        functools.partial(_sc_body, tc_v, n_rows),
        mesh=mesh,
        compiler_params=pltpu.CompilerParams(needs_layout_passes=False),
        out_type=(jax.ShapeDtypeStruct((NUM_WORKERS * n_rows,), jnp.float32),
                  jax.ShapeDtypeStruct((NUM_WORKERS * n_rows,), jnp.int32)),
        scratch_types=[
            pltpu.VMEM((2, SC_CHUNK, n_rows), jnp.float32),
            pltpu.VMEM((n_rows,), jnp.float32),
            pltpu.VMEM((n_rows,), jnp.int32),
            pltpu.SemaphoreType.DMA,
            pltpu.SemaphoreType.DMA,
        ],
    )
    sc_val, sc_idx = sc_run(lt)

    tc_val, tc_idx = pl.pallas_call(
        functools.partial(_tc_partial_body, n_blocks),
        grid=(n_blocks,),
        in_specs=[pl.BlockSpec((TC_BLOCK, n_rows), lambda v: (v, 0))],
        out_specs=[pl.BlockSpec((n_rows,), lambda v: (0,)),
                   pl.BlockSpec((n_rows,), lambda v: (0,))],
        out_shape=[jax.ShapeDtypeStruct((n_rows,), jnp.float32),
                   jax.ShapeDtypeStruct((n_rows,), jnp.int32)],
        scratch_shapes=[pltpu.VMEM((1, n_rows), jnp.float32),
                        pltpu.VMEM((1, n_rows), jnp.int32)],
        compiler_params=pltpu.CompilerParams(
            dimension_semantics=("arbitrary",)),
    )(lt)

    out = pl.pallas_call(
        functools.partial(_combine_body, n_rows, spec_len),
        out_shape=jax.ShapeDtypeStruct((n_rows,), jnp.int32),
    )(tc_val, tc_idx, sc_val, sc_idx, spec_pad)
    return out.reshape(batch, sample_len)
